# baseline (speedup 1.0000x reference)
import functools

import jax
import jax.numpy as jnp
from jax import lax
from jax.experimental import pallas as pl
from jax.experimental.pallas import tpu as pltpu
from jax.experimental.pallas import tpu_sc as plsc

NNZ = 1638400
VOCAB_SIZE = 1000000
KDIM = 16
NBATCH = 16384

NC = 2
NS = 16
NW = NC * NS
PER_W = NNZ // NW
CHUNK = 128
U = 8
NBODY = PER_W // (CHUNK * U)
GB = 128
NGB = CHUNK // GB
ROWS_PER_TILE = NBATCH // NS


def _sc_body(vals_hbm, fidx_hbm, rids_hbm, weight_hbm, out_hbm,
             fidx_v, rtmp_v, rids_v, vals_v, rows_v, wbuf_v, zbuf_v, acc_sh,
             psem, gsem0, gsem1, gsem2, gsem3, gsem4, gsem5, gsem6, gsem7,
             ssem0, ssem1, ssem2, ssem3, ssem4, ssem5, ssem6, ssem7):
    c_id = lax.axis_index("c")
    s_id = lax.axis_index("s")
    wid = c_id * NS + s_id
    base = wid * PER_W

    z16 = jnp.zeros((16,), jnp.float32)
    z16i = jnp.zeros((16,), jnp.int32)
    iota16 = lax.broadcasted_iota(jnp.int32, (16,), 0)
    iota16h = iota16 + 16
    gsems = (gsem0, gsem1, gsem2, gsem3, gsem4, gsem5, gsem6, gsem7)
    ssems = (ssem0, ssem1, ssem2, ssem3, ssem4, ssem5, ssem6, ssem7)

    def zero_zbuf(r, carry):
        plsc.store_scatter(zbuf_v, [z16i + r, iota16], z16)
        plsc.store_scatter(zbuf_v, [z16i + r, iota16h], z16)
        return carry

    lax.fori_loop(0, 128, zero_zbuf, 0)

    def zero_acc(t, carry):
        pltpu.sync_copy(zbuf_v, acc_sh.at[pl.ds(pl.multiple_of(s_id * ROWS_PER_TILE + t * 128, 128), 128)])
        return carry

    lax.fori_loop(0, ROWS_PER_TILE // 128, zero_acc, 0)
    plsc.subcore_barrier()

    def compute(u):
        def group_body(g, carry2):
            gb = u * CHUNK + g * 16
            v16 = vals_v[pl.ds(gb, 16)]
            ridx = z16i + gb
            for l in range(16):
                row = plsc.load_gather(rows_v, [ridx + l, iota16])
                w = row * v16[l]
                plsc.store_scatter(wbuf_v, [ridx + l, iota16], w)
                plsc.store_scatter(wbuf_v, [ridx + l, iota16h], w * w)
            return carry2

        lax.fori_loop(0, CHUNK // 16, group_body, 0)

    BODY_NNZ = U * CHUNK
    BODY_ROWS = BODY_NNZ // GB

    def issue_params(ii):
        cb = pl.multiple_of(base + ii * BODY_NNZ, BODY_NNZ)
        pltpu.async_copy(fidx_hbm.at[pl.ds(cb, BODY_NNZ)], fidx_v, psem)
        pltpu.async_copy(rids_hbm.at[pl.ds(cb, BODY_NNZ)], rtmp_v, psem)
        pltpu.async_copy(vals_hbm.at[pl.ds(cb, BODY_NNZ)], vals_v, psem)

    def wait_params():
        pltpu.make_async_copy(fidx_hbm.at[pl.ds(0, BODY_NNZ)], fidx_v, psem).wait()
        pltpu.make_async_copy(rids_hbm.at[pl.ds(0, BODY_NNZ)], rtmp_v, psem).wait()
        pltpu.make_async_copy(vals_hbm.at[pl.ds(0, BODY_NNZ)], vals_v, psem).wait()

    issue_params(0)

    def body(ii, carry):
        wait_params()
        for t in range(BODY_NNZ // 16):
            r16 = rtmp_v[pl.ds(t * 16, 16)]
            plsc.store_scatter(rids_v, [z16i + t // 8, iota16 + (t % 8) * 16], r16)
        gds = [
            [pltpu.async_copy(weight_hbm.at[fidx_v.at[pl.ds(u * CHUNK + j * GB, GB)]],
                              rows_v.at[pl.ds(u * CHUNK + j * GB, GB)],
                              gsems[u])
             for j in range(NGB)]
            for u in range(U)
        ]
        sds = []
        for u in range(U):
            for d in gds[u]:
                d.wait()
            compute(u)
            sds.append([
                pltpu.async_copy(wbuf_v.at[pl.ds(u * CHUNK + j * GB, GB)],
                                 acc_sh.at[rids_v.at[u * NGB + j]],
                                 ssems[u], add=True)
                for j in range(NGB)
            ])
            if u == U - 1:
                @pl.when(ii < NBODY - 1)
                def _next():
                    issue_params(ii + 1)
        for slot in sds:
            for d in slot:
                d.wait()
        return carry

    lax.fori_loop(0, NBODY, body, 0)

    plsc.subcore_barrier()
    out_base = pl.multiple_of(s_id * ROWS_PER_TILE, ROWS_PER_TILE)
    pltpu.sync_copy(acc_sh.at[pl.ds(out_base, ROWS_PER_TILE)],
                    out_hbm.at[c_id, pl.ds(out_base, ROWS_PER_TILE)])


_sc_kernel = functools.partial(
    pl.kernel,
    mesh=plsc.VectorSubcoreMesh(core_axis_name="c", subcore_axis_name="s",
                                num_cores=NC, num_subcores=NS),
    out_type=jax.ShapeDtypeStruct((NC, NBATCH, 32), jnp.float32),
    scratch_types=[
        pltpu.VMEM((U * CHUNK,), jnp.int32),
        pltpu.VMEM((U * CHUNK,), jnp.int32),
        pltpu.VMEM((U * NGB, GB), jnp.int32),
        pltpu.VMEM((U * CHUNK,), jnp.float32),
        pltpu.VMEM((U * CHUNK, KDIM), jnp.float32),
        pltpu.VMEM((U * CHUNK, 32), jnp.float32),
        pltpu.VMEM((128, 32), jnp.float32),
        pltpu.VMEM_SHARED((NBATCH, 32), jnp.float32),
    ] + [pltpu.SemaphoreType.DMA] * 17,
    compiler_params=pltpu.CompilerParams(needs_layout_passes=False, use_tc_tiling_on_sc=False),
)(_sc_body)


def _combine_body(p_ref, o_ref):
    x = p_ref[...]
    p = x[0] + x[1]
    k = lax.broadcasted_iota(jnp.int32, (NBATCH, 32), 1)
    t = jnp.where(k < KDIM, p * p, -p)
    o_ref[...] = jnp.sum(t, axis=1, keepdims=True)


_combine = pl.pallas_call(
    _combine_body,
    out_shape=jax.ShapeDtypeStruct((NBATCH, 1), jnp.float32),
)


def kernel(values, feat_idx, row_ids, weight):
    part = _sc_kernel(values, feat_idx, row_ids, weight)
    return _combine(part)


# speedup vs baseline: 1.0860x; 1.0860x over previous
import functools

import jax
import jax.numpy as jnp
from jax import lax
from jax.experimental import pallas as pl
from jax.experimental.pallas import tpu as pltpu
from jax.experimental.pallas import tpu_sc as plsc

NNZ = 1638400
VOCAB_SIZE = 1000000
KDIM = 16
NBATCH = 16384

NC = 2
NS = 16
NW = NC * NS
PER_W = NNZ // NW
CHUNK = 256
U = 4
NBODY = PER_W // (CHUNK * U)
GB = 128
NGB = CHUNK // GB
ROWS_PER_TILE = NBATCH // NS


def _sc_body(vals_hbm, fidx_hbm, rids_hbm, weight_hbm, out_hbm,
             fidx_v, rtmp_v, rids_v, vals_v, rows_v, wbuf_v, zbuf_v, acc_sh,
             psem, gsem0, gsem1, gsem2, gsem3, ssem0, ssem1, ssem2, ssem3):
    c_id = lax.axis_index("c")
    s_id = lax.axis_index("s")
    wid = c_id * NS + s_id
    base = wid * PER_W

    z16 = jnp.zeros((16,), jnp.float32)
    z16i = jnp.zeros((16,), jnp.int32)
    iota16 = lax.broadcasted_iota(jnp.int32, (16,), 0)
    iota16h = iota16 + 16
    gsems = (gsem0, gsem1, gsem2, gsem3)
    ssems = (ssem0, ssem1, ssem2, ssem3)

    def zero_zbuf(r, carry):
        plsc.store_scatter(zbuf_v, [z16i + r, iota16], z16)
        plsc.store_scatter(zbuf_v, [z16i + r, iota16h], z16)
        return carry

    lax.fori_loop(0, 128, zero_zbuf, 0)

    def zero_acc(t, carry):
        pltpu.sync_copy(zbuf_v, acc_sh.at[pl.ds(pl.multiple_of(s_id * ROWS_PER_TILE + t * 128, 128), 128)])
        return carry

    lax.fori_loop(0, ROWS_PER_TILE // 128, zero_acc, 0)
    plsc.subcore_barrier()

    def compute(u):
        @plsc.parallel_loop(0, CHUNK // 16, unroll=2)
        def group_body(g):
            gb = u * CHUNK + g * 16
            v16 = vals_v[pl.ds(gb, 16)]
            ridx = z16i + gb
            for l in range(16):
                row = plsc.load_gather(rows_v, [ridx + l, iota16])
                w = row * v16[l]
                plsc.store_scatter(wbuf_v, [ridx + l, iota16], w)
                plsc.store_scatter(wbuf_v, [ridx + l, iota16h], w * w)

    BODY_NNZ = U * CHUNK
    BODY_ROWS = BODY_NNZ // GB

    def issue_params(ii):
        cb = pl.multiple_of(base + ii * BODY_NNZ, BODY_NNZ)
        pltpu.async_copy(fidx_hbm.at[pl.ds(cb, BODY_NNZ)], fidx_v, psem)
        pltpu.async_copy(rids_hbm.at[pl.ds(cb, BODY_NNZ)], rtmp_v, psem)
        pltpu.async_copy(vals_hbm.at[pl.ds(cb, BODY_NNZ)], vals_v, psem)

    def wait_params():
        pltpu.make_async_copy(fidx_hbm.at[pl.ds(0, BODY_NNZ)], fidx_v, psem).wait()
        pltpu.make_async_copy(rids_hbm.at[pl.ds(0, BODY_NNZ)], rtmp_v, psem).wait()
        pltpu.make_async_copy(vals_hbm.at[pl.ds(0, BODY_NNZ)], vals_v, psem).wait()

    issue_params(0)

    def body(ii, carry):
        wait_params()
        for t in range(BODY_NNZ // 16):
            r16 = rtmp_v[pl.ds(t * 16, 16)]
            plsc.store_scatter(rids_v, [z16i + t // 8, iota16 + (t % 8) * 16], r16)
        gds = [
            [pltpu.async_copy(weight_hbm.at[fidx_v.at[pl.ds(u * CHUNK + j * GB, GB)]],
                              rows_v.at[pl.ds(u * CHUNK + j * GB, GB)],
                              gsems[u])
             for j in range(NGB)]
            for u in range(U)
        ]
        sds = []
        for u in range(U):
            for d in gds[u]:
                d.wait()
            compute(u)
            sds.append([
                pltpu.async_copy(wbuf_v.at[pl.ds(u * CHUNK + j * GB, GB)],
                                 acc_sh.at[rids_v.at[u * NGB + j]],
                                 ssems[u], add=True)
                for j in range(NGB)
            ])
            if u == U - 1:
                @pl.when(ii < NBODY - 1)
                def _next():
                    issue_params(ii + 1)
        for slot in sds:
            for d in slot:
                d.wait()
        return carry

    lax.fori_loop(0, NBODY, body, 0)

    plsc.subcore_barrier()
    out_base = pl.multiple_of(s_id * ROWS_PER_TILE, ROWS_PER_TILE)
    pltpu.sync_copy(acc_sh.at[pl.ds(out_base, ROWS_PER_TILE)],
                    out_hbm.at[c_id, pl.ds(out_base, ROWS_PER_TILE)])


_sc_kernel = functools.partial(
    pl.kernel,
    mesh=plsc.VectorSubcoreMesh(core_axis_name="c", subcore_axis_name="s",
                                num_cores=NC, num_subcores=NS),
    out_type=jax.ShapeDtypeStruct((NC, NBATCH, 32), jnp.float32),
    scratch_types=[
        pltpu.VMEM((U * CHUNK,), jnp.int32),
        pltpu.VMEM((U * CHUNK,), jnp.int32),
        pltpu.VMEM((U * NGB, GB), jnp.int32),
        pltpu.VMEM((U * CHUNK,), jnp.float32),
        pltpu.VMEM((U * CHUNK, KDIM), jnp.float32),
        pltpu.VMEM((U * CHUNK, 32), jnp.float32),
        pltpu.VMEM((128, 32), jnp.float32),
        pltpu.VMEM_SHARED((NBATCH, 32), jnp.float32),
    ] + [pltpu.SemaphoreType.DMA] * 9,
    compiler_params=pltpu.CompilerParams(needs_layout_passes=False, use_tc_tiling_on_sc=False),
)(_sc_body)


def _combine_body(p_ref, o_ref):
    x = p_ref[...]
    p = x[0] + x[1]
    k = lax.broadcasted_iota(jnp.int32, (NBATCH, 32), 1)
    t = jnp.where(k < KDIM, p * p, -p)
    o_ref[...] = jnp.sum(t, axis=1, keepdims=True)


_combine = pl.pallas_call(
    _combine_body,
    out_shape=jax.ShapeDtypeStruct((NBATCH, 1), jnp.float32),
)


def kernel(values, feat_idx, row_ids, weight):
    part = _sc_kernel(values, feat_idx, row_ids, weight)
    return _combine(part)
